# Initial kernel scaffold; baseline (speedup 1.0000x reference)
import jax
import jax.numpy as jnp
from jax.experimental import pallas as pl


def kernel(queries, keys, values, attn_mask):
    raise NotImplementedError("write your pallas kernel here")


# speedup vs baseline: 4.5435x; 4.5435x over previous
import functools
import numpy as np
import jax
import jax.numpy as jnp
from jax.experimental import pallas as pl

L = 1024
F = 520
TOPK = 6
NEG = -3.0e38


def _dft_mats():
    l = np.arange(L, dtype=np.int64)
    f = np.arange(F, dtype=np.int64)
    m = (np.outer(f, l) % L).astype(np.float64) * (2.0 * np.pi / L)
    c = np.cos(m)
    s = np.sin(m)
    valid = (f <= L // 2).astype(np.float64)[:, None]
    wc = (c * valid).astype(np.float32)
    ws = (-s * valid).astype(np.float32)
    wf = np.where((f == 0) | (f == L // 2), 1.0, 2.0) / L
    ic = (c.T * wf[None, :] * valid.T).astype(np.float32)
    isn = (-s.T * wf[None, :] * valid.T).astype(np.float32)
    return wc, ws, ic, isn


_WC, _WS, _IC, _ISN = _dft_mats()


def _body(q_ref, k_ref, v_ref, wc_ref, ws_ref, ic_ref, isn_ref,
          corr_ref, vout_ref):
    C = q_ref.shape[-1]
    dot = functools.partial(
        jax.lax.dot_general,
        dimension_numbers=(((1,), (0,)), ((), ())),
        preferred_element_type=jnp.float32,
        precision=jax.lax.Precision.HIGHEST)
    q = q_ref[0]
    k = k_ref[0]
    v = v_ref[0]
    wc = wc_ref[...]
    ws = ws_ref[...]
    ic = ic_ref[...]
    isn = isn_ref[...]

    qr = dot(wc, q)
    qi = dot(ws, q)
    kr = dot(wc, k)
    ki = dot(ws, k)
    pr = qr * kr + qi * ki
    pi = qi * kr - qr * ki
    corr = dot(ic, pr) + dot(isn, pi)
    corr_ref[0] = corr

    riota = jax.lax.broadcasted_iota(jnp.int32, (L, C), 0)
    c = corr
    tops, delays = [], []
    for _ in range(TOPK):
        m = jnp.max(c, axis=0, keepdims=True)
        idx = jnp.min(jnp.where(c == m, riota, L), axis=0, keepdims=True)
        c = jnp.where(riota == idx, NEG, c)
        tops.append(m)
        delays.append(idx)

    es = [jnp.exp(w - tops[0]) for w in tops]
    tot = es[0]
    for e in es[1:]:
        tot = tot + e

    h = jnp.zeros((L, C), jnp.float32)
    for e, d in zip(es, delays):
        h = jnp.where(riota == d, e / tot, h)

    hr = dot(wc, h)
    hi = dot(ws, h)
    vr = dot(wc, v)
    vi = dot(ws, v)
    vfr = vr * hr + vi * hi
    vfi = vi * hr - vr * hi
    vout_ref[0] = dot(ic, vfr) + dot(isn, vfi)


def kernel(queries, keys, values, attn_mask):
    B, N, Lq, H, E = queries.shape
    C = H * E
    BN = B * N
    q = queries.reshape(BN, Lq, C)
    k = keys.reshape(BN, Lq, C)
    v = values.reshape(BN, Lq, C)
    wc = jnp.asarray(_WC)
    ws = jnp.asarray(_WS)
    ic = jnp.asarray(_IC)
    isn = jnp.asarray(_ISN)

    CB = 128
    blk = lambda i, j: (i, 0, j)
    fix = lambda i, j: (0, 0)
    corr, vout = pl.pallas_call(
        _body,
        grid=(BN, C // CB),
        in_specs=[
            pl.BlockSpec((1, L, CB), blk),
            pl.BlockSpec((1, L, CB), blk),
            pl.BlockSpec((1, L, CB), blk),
            pl.BlockSpec((F, L), fix),
            pl.BlockSpec((F, L), fix),
            pl.BlockSpec((L, F), fix),
            pl.BlockSpec((L, F), fix),
        ],
        out_specs=[
            pl.BlockSpec((1, L, CB), blk),
            pl.BlockSpec((1, L, CB), blk),
        ],
        out_shape=[
            jax.ShapeDtypeStruct((BN, L, C), jnp.float32),
            jax.ShapeDtypeStruct((BN, L, C), jnp.float32),
        ],
    )(q, k, v, wc, ws, ic, isn)

    V = vout.reshape(B, N, Lq, H, E)
    corr_t = corr.reshape(B, N, Lq, H, E)
    return (V, corr_t)


# speedup vs baseline: 7.9431x; 1.7482x over previous
import functools
import numpy as np
import jax
import jax.numpy as jnp
from jax.experimental import pallas as pl

L = 1024
F = 520
TOPK = 6
NEG = -3.0e38


def _dft_mats():
    l = np.arange(L, dtype=np.int64)
    f = np.arange(F, dtype=np.int64)
    m = (np.outer(f, l) % L).astype(np.float64) * (2.0 * np.pi / L)
    c = np.cos(m)
    s = np.sin(m)
    valid = (f <= L // 2).astype(np.float64)[:, None]
    wc = (c * valid).astype(np.float32)
    ws = (-s * valid).astype(np.float32)
    wf = np.where((f == 0) | (f == L // 2), 1.0, 2.0) / L
    ic = (c.T * wf[None, :] * valid.T).astype(np.float32)
    isn = (-s.T * wf[None, :] * valid.T).astype(np.float32)
    return wc, ws, ic, isn


_WC, _WS, _IC, _ISN = _dft_mats()


def _split(x):
    hi = x.astype(jnp.bfloat16)
    lo = (x - hi.astype(jnp.float32)).astype(jnp.bfloat16)
    return hi, lo


def _body(q_ref, k_ref, v_ref, wc_ref, ws_ref, ic_ref, isn_ref,
          corr_ref, vout_ref):
    C = q_ref.shape[-1]
    dot = functools.partial(
        jax.lax.dot_general,
        dimension_numbers=(((1,), (0,)), ((), ())),
        preferred_element_type=jnp.float32)

    def dot3(a, b):
        ah, al = _split(a)
        bh, bl = _split(b)
        return dot(ah, bh) + (dot(ah, bl) + dot(al, bh))

    q = q_ref[0]
    k = k_ref[0]
    v = v_ref[0]
    wc = wc_ref[...]
    ws = ws_ref[...]
    ic = ic_ref[...]
    isn = isn_ref[...]

    qr = dot3(wc, q)
    qi = dot3(ws, q)
    kr = dot3(wc, k)
    ki = dot3(ws, k)
    pr = qr * kr + qi * ki
    pi = qi * kr - qr * ki
    corr = dot3(ic, pr) + dot3(isn, pi)
    corr_ref[0] = corr

    riota = jax.lax.broadcasted_iota(jnp.int32, (L, C), 0)
    c = corr
    tops, delays = [], []
    for _ in range(TOPK):
        m = jnp.max(c, axis=0, keepdims=True)
        idx = jnp.min(jnp.where(c == m, riota, L), axis=0, keepdims=True)
        c = jnp.where(riota == idx, NEG, c)
        tops.append(m)
        delays.append(idx)

    es = [jnp.exp(w - tops[0]) for w in tops]
    tot = es[0]
    for e in es[1:]:
        tot = tot + e

    h = jnp.zeros((L, C), jnp.float32)
    for e, d in zip(es, delays):
        h = jnp.where(riota == d, e / tot, h)

    hr = dot3(wc, h)
    hi = dot3(ws, h)
    vr = dot3(wc, v)
    vi = dot3(ws, v)
    vfr = vr * hr + vi * hi
    vfi = vi * hr - vr * hi
    vout_ref[0] = dot3(ic, vfr) + dot3(isn, vfi)


def kernel(queries, keys, values, attn_mask):
    B, N, Lq, H, E = queries.shape
    C = H * E
    BN = B * N
    q = queries.reshape(BN, Lq, C)
    k = keys.reshape(BN, Lq, C)
    v = values.reshape(BN, Lq, C)
    wc = jnp.asarray(_WC)
    ws = jnp.asarray(_WS)
    ic = jnp.asarray(_IC)
    isn = jnp.asarray(_ISN)

    CB = 128
    blk = lambda i, j: (i, 0, j)
    fix = lambda i, j: (0, 0)
    corr, vout = pl.pallas_call(
        _body,
        grid=(BN, C // CB),
        in_specs=[
            pl.BlockSpec((1, L, CB), blk),
            pl.BlockSpec((1, L, CB), blk),
            pl.BlockSpec((1, L, CB), blk),
            pl.BlockSpec((F, L), fix),
            pl.BlockSpec((F, L), fix),
            pl.BlockSpec((L, F), fix),
            pl.BlockSpec((L, F), fix),
        ],
        out_specs=[
            pl.BlockSpec((1, L, CB), blk),
            pl.BlockSpec((1, L, CB), blk),
        ],
        out_shape=[
            jax.ShapeDtypeStruct((BN, L, C), jnp.float32),
            jax.ShapeDtypeStruct((BN, L, C), jnp.float32),
        ],
    )(q, k, v, wc, ws, ic, isn)

    V = vout.reshape(B, N, Lq, H, E)
    corr_t = corr.reshape(B, N, Lq, H, E)
    return (V, corr_t)


# speedup vs baseline: 8.3794x; 1.0549x over previous
import functools
import numpy as np
import jax
import jax.numpy as jnp
from jax import lax
from jax.experimental import pallas as pl
from jax.experimental.pallas import tpu as pltpu
from jax.experimental.pallas import tpu_sc as plsc

L = 1024
F = 520
TOPK = 6
NEG = -3.0e38


def _dft_mats():
    l = np.arange(L, dtype=np.int64)
    f = np.arange(F, dtype=np.int64)
    m = (np.outer(f, l) % L).astype(np.float64) * (2.0 * np.pi / L)
    c = np.cos(m)
    s = np.sin(m)
    valid = (f <= L // 2).astype(np.float64)[:, None]
    wc = (c * valid).astype(np.float32)
    ws = (-s * valid).astype(np.float32)
    wf = np.where((f == 0) | (f == L // 2), 1.0, 2.0) / L
    ic = (c.T * wf[None, :] * valid.T).astype(np.float32)
    isn = (-s.T * wf[None, :] * valid.T).astype(np.float32)
    return wc, ws, ic, isn


_WC, _WS, _IC, _ISN = _dft_mats()


def _split(x):
    hi = x.astype(jnp.bfloat16)
    lo = (x - hi.astype(jnp.float32)).astype(jnp.bfloat16)
    return hi, lo


def _corr_body(q_ref, k_ref, wc_ref, ws_ref, ic_ref, isn_ref,
               corr_ref, w_ref, d_ref):
    C = q_ref.shape[-1]
    dot = functools.partial(
        jax.lax.dot_general,
        dimension_numbers=(((1,), (0,)), ((), ())),
        preferred_element_type=jnp.float32)

    def dot3(a, b):
        ah, al = _split(a)
        bh, bl = _split(b)
        return dot(ah, bh) + (dot(ah, bl) + dot(al, bh))

    q = q_ref[0]
    k = k_ref[0]
    wc = wc_ref[...]
    ws = ws_ref[...]

    qr = dot3(wc, q)
    qi = dot3(ws, q)
    kr = dot3(wc, k)
    ki = dot3(ws, k)
    pr = qr * kr + qi * ki
    pi = qi * kr - qr * ki
    corr = dot3(ic_ref[...], pr) + dot3(isn_ref[...], pi)
    corr_ref[0] = corr

    riota = jax.lax.broadcasted_iota(jnp.int32, (L, C), 0)
    c = corr
    tops, delays = [], []
    for _ in range(TOPK):
        m = jnp.max(c, axis=0, keepdims=True)
        idx = jnp.min(jnp.where(c == m, riota, L), axis=0, keepdims=True)
        c = jnp.where(riota == idx, NEG, c)
        tops.append(m)
        delays.append(idx)

    es = [jnp.exp(w - tops[0]) for w in tops]
    tot = es[0]
    for e in es[1:]:
        tot = tot + e
    inv = 1.0 / tot

    zero_f = jnp.zeros((2, C), jnp.float32)
    zero_i = jnp.zeros((2, C), jnp.int32)
    w_ref[0] = jnp.concatenate([e * inv for e in es] + [zero_f], axis=0)
    d_ref[0] = jnp.concatenate(delays + [zero_i], axis=0)


def _corr_topk(q, k):
    BN, Lq, C = q.shape
    CB = 128
    blk = lambda i, j: (i, 0, j)
    fix = lambda i, j: (0, 0)
    return pl.pallas_call(
        _corr_body,
        grid=(BN, C // CB),
        in_specs=[
            pl.BlockSpec((1, L, CB), blk),
            pl.BlockSpec((1, L, CB), blk),
            pl.BlockSpec((F, L), fix),
            pl.BlockSpec((F, L), fix),
            pl.BlockSpec((L, F), fix),
            pl.BlockSpec((L, F), fix),
        ],
        out_specs=[
            pl.BlockSpec((1, L, CB), blk),
            pl.BlockSpec((1, 8, CB), blk),
            pl.BlockSpec((1, 8, CB), blk),
        ],
        out_shape=[
            jax.ShapeDtypeStruct((BN, L, C), jnp.float32),
            jax.ShapeDtypeStruct((BN, 8, C), jnp.float32),
            jax.ShapeDtypeStruct((BN, 8, C), jnp.int32),
        ],
    )(q, k, jnp.asarray(_WC), jnp.asarray(_WS),
      jnp.asarray(_IC), jnp.asarray(_ISN))


def _delay_agg(v, w, d):
    BN, Lq, C = v.shape
    info = plsc.get_sparse_core_info()
    NC, NS, NL = info.num_cores, info.num_subcores, info.num_lanes
    NW = NC * NS
    n_chunks = C // NL
    n_tasks = BN * n_chunks
    tasks_per_w = n_tasks // NW
    mesh = plsc.VectorSubcoreMesh(core_axis_name="c", subcore_axis_name="s")

    @functools.partial(
        pl.kernel,
        mesh=mesh,
        compiler_params=pltpu.CompilerParams(
            use_tc_tiling_on_sc=False, needs_layout_passes=False),
        out_type=jax.ShapeDtypeStruct((BN, Lq, C), jnp.float32),
        scratch_types=[
            pltpu.VMEM((Lq, NL), jnp.float32),
            pltpu.VMEM((8, NL), jnp.float32),
            pltpu.VMEM((8, NL), jnp.int32),
            pltpu.VMEM((Lq, NL), jnp.float32),
        ],
    )
    def agg(v_hbm, w_hbm, d_hbm, out_hbm, vbuf, wbuf, dbuf, obuf):
        wid = lax.axis_index("s") * NC + lax.axis_index("c")
        lanes = jax.lax.broadcasted_iota(jnp.int32, (NL,), 0)
        for t in range(tasks_per_w):
            task = wid * tasks_per_w + t
            bn = task // n_chunks
            ch0 = (task % n_chunks) * NL
            pltpu.sync_copy(v_hbm.at[bn, :, pl.ds(ch0, NL)], vbuf)
            pltpu.sync_copy(w_hbm.at[bn, :, pl.ds(ch0, NL)], wbuf)
            pltpu.sync_copy(d_hbm.at[bn, :, pl.ds(ch0, NL)], dbuf)
            wv = [wbuf[i] for i in range(TOPK)]
            dv = [dbuf[i] for i in range(TOPK)]

            def row(l, carry):
                for u in range(8):
                    ll = l * 8 + u
                    acc = None
                    for i in range(TOPK):
                        idx = jnp.bitwise_and(dv[i] + ll, L - 1)
                        g = plsc.load_gather(vbuf, [idx, lanes])
                        acc = g * wv[i] if acc is None else acc + g * wv[i]
                    obuf[ll] = acc
                return carry

            lax.fori_loop(0, Lq // 8, row, 0)
            pltpu.sync_copy(obuf, out_hbm.at[bn, :, pl.ds(ch0, NL)])

    return agg(v, w, d)


def kernel(queries, keys, values, attn_mask):
    B, N, Lq, H, E = queries.shape
    C = H * E
    BN = B * N
    q = queries.reshape(BN, Lq, C)
    k = keys.reshape(BN, Lq, C)
    v = values.reshape(BN, Lq, C)

    corr, w, d = _corr_topk(q, k)
    vout = _delay_agg(v, w, d)

    V = vout.reshape(B, N, Lq, H, E)
    corr_t = corr.reshape(B, N, Lq, H, E)
    return (V, corr_t)


# speedup vs baseline: 10.4634x; 1.2487x over previous
import functools
import numpy as np
import jax
import jax.numpy as jnp
from jax import lax
from jax.experimental import pallas as pl
from jax.experimental.pallas import tpu as pltpu
from jax.experimental.pallas import tpu_sc as plsc

L = 1024
F = 520
TOPK = 6
NEG = -3.0e38


def _dft_mats():
    l = np.arange(L, dtype=np.int64)
    f = np.arange(F, dtype=np.int64)
    m = (np.outer(f, l) % L).astype(np.float64) * (2.0 * np.pi / L)
    c = np.cos(m)
    s = np.sin(m)
    valid = (f <= L // 2).astype(np.float64)[:, None]
    wc = (c * valid).astype(np.float32)
    ws = (-s * valid).astype(np.float32)
    wf = np.where((f == 0) | (f == L // 2), 1.0, 2.0) / L
    ic = (c.T * wf[None, :] * valid.T).astype(np.float32)
    isn = (-s.T * wf[None, :] * valid.T).astype(np.float32)
    return wc, ws, ic, isn


_WC, _WS, _IC, _ISN = _dft_mats()


def _split(x):
    hi = x.astype(jnp.bfloat16)
    lo = (x - hi.astype(jnp.float32)).astype(jnp.bfloat16)
    return hi, lo


def _corr_body(q_ref, k_ref, wc_ref, ws_ref, ic_ref, isn_ref,
               corr_ref, w_ref, d_ref):
    C = q_ref.shape[-1]
    dot = functools.partial(
        jax.lax.dot_general,
        dimension_numbers=(((1,), (0,)), ((), ())),
        preferred_element_type=jnp.float32)

    def dot3(a, b):
        ah, al = _split(a)
        bh, bl = _split(b)
        return dot(ah, bh) + (dot(ah, bl) + dot(al, bh))

    q = q_ref[0]
    k = k_ref[0]
    wc = wc_ref[...]
    ws = ws_ref[...]

    qr = dot3(wc, q)
    qi = dot3(ws, q)
    kr = dot3(wc, k)
    ki = dot3(ws, k)
    pr = qr * kr + qi * ki
    pi = qi * kr - qr * ki
    corr = dot3(ic_ref[...], pr) + dot3(isn_ref[...], pi)
    corr_ref[0] = corr

    riota = jax.lax.broadcasted_iota(jnp.int32, (L, C), 0)
    c = corr
    tops, delays = [], []
    for _ in range(TOPK):
        m = jnp.max(c, axis=0, keepdims=True)
        idx = jnp.min(jnp.where(c == m, riota, L), axis=0, keepdims=True)
        c = jnp.where(riota == idx, NEG, c)
        tops.append(m)
        delays.append(idx)

    es = [jnp.exp(w - tops[0]) for w in tops]
    tot = es[0]
    for e in es[1:]:
        tot = tot + e
    inv = 1.0 / tot

    zero_f = jnp.zeros((2, C), jnp.float32)
    zero_i = jnp.zeros((2, C), jnp.int32)
    w_ref[0] = jnp.concatenate([e * inv for e in es] + [zero_f], axis=0)
    d_ref[0] = jnp.concatenate(delays + [zero_i], axis=0)


def _corr_topk(q, k):
    BN, Lq, C = q.shape
    CB = 256
    blk = lambda i, j: (i, 0, j)
    fix = lambda i, j: (0, 0)
    return pl.pallas_call(
        _corr_body,
        grid=(BN, C // CB),
        in_specs=[
            pl.BlockSpec((1, L, CB), blk),
            pl.BlockSpec((1, L, CB), blk),
            pl.BlockSpec((F, L), fix),
            pl.BlockSpec((F, L), fix),
            pl.BlockSpec((L, F), fix),
            pl.BlockSpec((L, F), fix),
        ],
        out_specs=[
            pl.BlockSpec((1, L, CB), blk),
            pl.BlockSpec((1, 8, CB), blk),
            pl.BlockSpec((1, 8, CB), blk),
        ],
        out_shape=[
            jax.ShapeDtypeStruct((BN, L, C), jnp.float32),
            jax.ShapeDtypeStruct((BN, 8, C), jnp.float32),
            jax.ShapeDtypeStruct((BN, 8, C), jnp.int32),
        ],
    )(q, k, jnp.asarray(_WC), jnp.asarray(_WS),
      jnp.asarray(_IC), jnp.asarray(_ISN))


def _delay_agg(v, w, d):
    BN, Lq, C = v.shape
    info = plsc.get_sparse_core_info()
    NC, NS, NL = info.num_cores, info.num_subcores, info.num_lanes
    NW = NC * NS
    n_chunks = C // NL
    n_tasks = BN * n_chunks
    tasks_per_w = n_tasks // NW
    mesh = plsc.VectorSubcoreMesh(core_axis_name="c", subcore_axis_name="s")

    @functools.partial(
        pl.kernel,
        mesh=mesh,
        compiler_params=pltpu.CompilerParams(
            use_tc_tiling_on_sc=False, needs_layout_passes=False),
        out_type=jax.ShapeDtypeStruct((BN, Lq, C), jnp.float32),
        scratch_types=[
            pltpu.VMEM((Lq, NL), jnp.float32),
            pltpu.VMEM((8, NL), jnp.float32),
            pltpu.VMEM((8, NL), jnp.int32),
            pltpu.VMEM((Lq, NL), jnp.float32),
        ],
    )
    def agg(v_hbm, w_hbm, d_hbm, out_hbm, vbuf, wbuf, dbuf, obuf):
        wid = lax.axis_index("s") * NC + lax.axis_index("c")
        lanes = jax.lax.broadcasted_iota(jnp.int32, (NL,), 0)
        for t in range(tasks_per_w):
            task = wid * tasks_per_w + t
            bn = task // n_chunks
            ch0 = (task % n_chunks) * NL
            pltpu.sync_copy(v_hbm.at[bn, :, pl.ds(ch0, NL)], vbuf)
            pltpu.sync_copy(w_hbm.at[bn, :, pl.ds(ch0, NL)], wbuf)
            pltpu.sync_copy(d_hbm.at[bn, :, pl.ds(ch0, NL)], dbuf)
            wv = [wbuf[i] for i in range(TOPK)]
            dv = [dbuf[i] for i in range(TOPK)]

            def row(l, carry):
                for u in range(8):
                    ll = l * 8 + u
                    acc = None
                    for i in range(TOPK):
                        idx = jnp.bitwise_and(dv[i] + ll, L - 1)
                        g = plsc.load_gather(vbuf, [idx, lanes])
                        acc = g * wv[i] if acc is None else acc + g * wv[i]
                    obuf[ll] = acc
                return carry

            lax.fori_loop(0, Lq // 8, row, 0)
            pltpu.sync_copy(obuf, out_hbm.at[bn, :, pl.ds(ch0, NL)])

    return agg(v, w, d)


def kernel(queries, keys, values, attn_mask):
    B, N, Lq, H, E = queries.shape
    C = H * E
    BN = B * N
    q = queries.reshape(BN, Lq, C)
    k = keys.reshape(BN, Lq, C)
    v = values.reshape(BN, Lq, C)

    corr, w, d = _corr_topk(q, k)
    vout = _delay_agg(v, w, d)

    V = vout.reshape(B, N, Lq, H, E)
    corr_t = corr.reshape(B, N, Lq, H, E)
    return (V, corr_t)


# speedup vs baseline: 10.5766x; 1.0108x over previous
import functools
import numpy as np
import jax
import jax.numpy as jnp
from jax import lax
from jax.experimental import pallas as pl
from jax.experimental.pallas import tpu as pltpu
from jax.experimental.pallas import tpu_sc as plsc

L = 1024
F = 520
TOPK = 6
NEG = -3.0e38


def _dft_mats():
    l = np.arange(L, dtype=np.int64)
    f = np.arange(F, dtype=np.int64)
    m = (np.outer(f, l) % L).astype(np.float64) * (2.0 * np.pi / L)
    c = np.cos(m)
    s = np.sin(m)
    valid = (f <= L // 2).astype(np.float64)[:, None]
    wc = (c * valid).astype(np.float32)
    ws = (-s * valid).astype(np.float32)
    wf = np.where((f == 0) | (f == L // 2), 1.0, 2.0) / L
    ic = (c.T * wf[None, :] * valid.T).astype(np.float32)
    isn = (-s.T * wf[None, :] * valid.T).astype(np.float32)
    return wc, ws, ic, isn


_WC, _WS, _IC, _ISN = _dft_mats()


def _split(x):
    hi = x.astype(jnp.bfloat16)
    lo = (x - hi.astype(jnp.float32)).astype(jnp.bfloat16)
    return hi, lo


def _corr_body(q_ref, k_ref, wc_ref, ws_ref, ic_ref, isn_ref,
               corr_ref, w_ref, d_ref):
    C = q_ref.shape[-1]
    dot = functools.partial(
        jax.lax.dot_general,
        dimension_numbers=(((1,), (0,)), ((), ())),
        preferred_element_type=jnp.float32)

    def dot3(a, b):
        ah, al = _split(a)
        bh, bl = _split(b)
        return dot(ah, bh) + (dot(ah, bl) + dot(al, bh))

    q = q_ref[0]
    k = k_ref[0]
    wc = wc_ref[...]
    ws = ws_ref[...]

    qr = dot3(wc, q)
    qi = dot3(ws, q)
    kr = dot3(wc, k)
    ki = dot3(ws, k)
    pr = qr * kr + qi * ki
    pi = qi * kr - qr * ki
    corr = dot3(ic_ref[...], pr) + dot3(isn_ref[...], pi)
    corr_ref[0] = corr

    riota = jax.lax.broadcasted_iota(jnp.int32, (L, C), 0)
    c = corr
    tops, delays = [], []
    for _ in range(TOPK):
        m = jnp.max(c, axis=0, keepdims=True)
        idx = jnp.min(jnp.where(c == m, riota, L), axis=0, keepdims=True)
        c = jnp.where(riota == idx, NEG, c)
        tops.append(m)
        delays.append(idx)

    es = [jnp.exp(w - tops[0]) for w in tops]
    tot = es[0]
    for e in es[1:]:
        tot = tot + e
    inv = 1.0 / tot

    zero_f = jnp.zeros((2, C), jnp.float32)
    zero_i = jnp.zeros((2, C), jnp.int32)
    w_ref[0] = jnp.concatenate([e * inv for e in es] + [zero_f], axis=0)
    d_ref[0] = jnp.concatenate(delays + [zero_i], axis=0)


def _corr_topk(q, k):
    BN, Lq, C = q.shape
    CB = 512
    blk = lambda i, j: (i, 0, j)
    fix = lambda i, j: (0, 0)
    return pl.pallas_call(
        _corr_body,
        grid=(BN, C // CB),
        in_specs=[
            pl.BlockSpec((1, L, CB), blk),
            pl.BlockSpec((1, L, CB), blk),
            pl.BlockSpec((F, L), fix),
            pl.BlockSpec((F, L), fix),
            pl.BlockSpec((L, F), fix),
            pl.BlockSpec((L, F), fix),
        ],
        out_specs=[
            pl.BlockSpec((1, L, CB), blk),
            pl.BlockSpec((1, 8, CB), blk),
            pl.BlockSpec((1, 8, CB), blk),
        ],
        out_shape=[
            jax.ShapeDtypeStruct((BN, L, C), jnp.float32),
            jax.ShapeDtypeStruct((BN, 8, C), jnp.float32),
            jax.ShapeDtypeStruct((BN, 8, C), jnp.int32),
        ],
    )(q, k, jnp.asarray(_WC), jnp.asarray(_WS),
      jnp.asarray(_IC), jnp.asarray(_ISN))


def _delay_agg(v, w, d):
    BN, Lq, C = v.shape
    info = plsc.get_sparse_core_info()
    NC, NS, NL = info.num_cores, info.num_subcores, info.num_lanes
    NW = NC * NS
    n_chunks = C // NL
    n_tasks = BN * n_chunks
    tasks_per_w = n_tasks // NW
    mesh = plsc.VectorSubcoreMesh(core_axis_name="c", subcore_axis_name="s")

    @functools.partial(
        pl.kernel,
        mesh=mesh,
        compiler_params=pltpu.CompilerParams(
            use_tc_tiling_on_sc=False, needs_layout_passes=False),
        out_type=jax.ShapeDtypeStruct((BN, Lq, C), jnp.float32),
        scratch_types=[
            pltpu.VMEM((Lq, NL), jnp.float32),
            pltpu.VMEM((8, NL), jnp.float32),
            pltpu.VMEM((8, NL), jnp.int32),
            pltpu.VMEM((Lq, NL), jnp.float32),
        ],
    )
    def agg(v_hbm, w_hbm, d_hbm, out_hbm, vbuf, wbuf, dbuf, obuf):
        wid = lax.axis_index("s") * NC + lax.axis_index("c")
        lanes = jax.lax.broadcasted_iota(jnp.int32, (NL,), 0)
        for t in range(tasks_per_w):
            task = wid * tasks_per_w + t
            bn = task // n_chunks
            ch0 = (task % n_chunks) * NL
            pltpu.sync_copy(v_hbm.at[bn, :, pl.ds(ch0, NL)], vbuf)
            pltpu.sync_copy(w_hbm.at[bn, :, pl.ds(ch0, NL)], wbuf)
            pltpu.sync_copy(d_hbm.at[bn, :, pl.ds(ch0, NL)], dbuf)
            wv = [wbuf[i] for i in range(TOPK)]
            dv = [dbuf[i] for i in range(TOPK)]

            def row(l, carry):
                for u in range(8):
                    ll = l * 8 + u
                    acc = None
                    for i in range(TOPK):
                        idx = jnp.bitwise_and(dv[i] + ll, L - 1)
                        g = plsc.load_gather(vbuf, [idx, lanes])
                        acc = g * wv[i] if acc is None else acc + g * wv[i]
                    obuf[ll] = acc
                return carry

            lax.fori_loop(0, Lq // 8, row, 0)
            pltpu.sync_copy(obuf, out_hbm.at[bn, :, pl.ds(ch0, NL)])

    return agg(v, w, d)


def kernel(queries, keys, values, attn_mask):
    B, N, Lq, H, E = queries.shape
    C = H * E
    BN = B * N
    q = queries.reshape(BN, Lq, C)
    k = keys.reshape(BN, Lq, C)
    v = values.reshape(BN, Lq, C)

    corr, w, d = _corr_topk(q, k)
    vout = _delay_agg(v, w, d)

    V = vout.reshape(B, N, Lq, H, E)
    corr_t = corr.reshape(B, N, Lq, H, E)
    return (V, corr_t)
